# Initial kernel scaffold; baseline (speedup 1.0000x reference)
import functools

import jax
import jax.numpy as jnp
from jax import lax
from jax.experimental import pallas as pl
from jax.experimental.pallas import tpu as pltpu
from jax.experimental.pallas import tpu_sc as plsc

B, N, DIM = 8, 1024, 32
K = 8192
M = B * N
TN = 256
NUM_TILES = M // TN
LOSS_SCALE = 1.25 / (M * DIM)


def _argmin_body(x_ref, frozen_ref, w_ref, ic_ref, idx_ref, loss_ref, icat_ref):
    pi = pl.program_id(0)

    @pl.when(pi == 0)
    def _():
        ic = jnp.dot(frozen_ref[...], w_ref[...],
                     preferred_element_type=jnp.float32)
        ic_ref[...] = ic
        ict = ic.T
        c2 = jnp.sum(ict * ict, axis=0, keepdims=True)
        icat_ref[:DIM, :] = ict * -2.0
        icat_ref[DIM:DIM + 1, :] = c2

    xt = x_ref[...]
    xa = jnp.concatenate([xt, jnp.ones((TN, 1), jnp.float32)], axis=1)
    s = jnp.dot(xa, icat_ref[...], preferred_element_type=jnp.float32)
    minv = jnp.min(s, axis=1, keepdims=True)
    iota = lax.broadcasted_iota(jnp.int32, (TN, K), 1)
    idx = jnp.min(jnp.where(s == minv, iota, K), axis=1)
    idx_ref[0, 0, :] = idx
    x2 = jnp.sum(xt * xt, axis=1)
    part = jnp.sum(jnp.maximum(x2 + minv[:, 0], 0.0))

    @pl.when(pi == 0)
    def _():
        loss_ref[0, 0] = 0.0

    loss_ref[0, 0] += part


_argmin_call = pl.pallas_call(
    _argmin_body,
    grid=(NUM_TILES,),
    in_specs=[
        pl.BlockSpec((TN, DIM), lambda i: (i, 0)),
        pl.BlockSpec((K, DIM), lambda i: (0, 0)),
        pl.BlockSpec((DIM, DIM), lambda i: (0, 0)),
    ],
    out_specs=[
        pl.BlockSpec((K, DIM), lambda i: (0, 0)),
        pl.BlockSpec((1, 1, TN), lambda i: (i, 0, 0)),
        pl.BlockSpec((1, 1), lambda i: (0, 0)),
    ],
    out_shape=[
        jax.ShapeDtypeStruct((K, DIM), jnp.float32),
        jax.ShapeDtypeStruct((NUM_TILES, 1, TN), jnp.int32),
        jax.ShapeDtypeStruct((1, 1), jnp.float32),
    ],
    scratch_shapes=[pltpu.VMEM((DIM + 1, K), jnp.float32)],
)


def _sc_gather(table, idx):
    info = plsc.get_sparse_core_info()
    nw = info.num_cores * info.num_subcores
    bpw = M // nw
    mesh = plsc.VectorSubcoreMesh(core_axis_name="c", subcore_axis_name="s")

    @functools.partial(
        pl.kernel, mesh=mesh,
        out_type=jax.ShapeDtypeStruct((M, DIM), jnp.float32),
        scratch_types=[
            pltpu.VMEM((bpw,), jnp.int32),
            pltpu.VMEM((bpw, DIM), jnp.float32),
            pltpu.SemaphoreType.DMA,
        ],
    )
    def k(table_hbm, idx_hbm, out_hbm, idx_v, rows_v, sem):
        wid = lax.axis_index("s") * info.num_cores + lax.axis_index("c")
        base = wid * bpw
        pltpu.sync_copy(idx_hbm.at[pl.ds(base, bpw)], idx_v)
        pltpu.async_copy(table_hbm.at[idx_v], rows_v, sem).wait()
        pltpu.sync_copy(rows_v, out_hbm.at[pl.ds(base, bpw)])

    return k(table, idx)


def kernel(x, frozen_codebook, W):
    xf = x.reshape(M, DIM)
    ic, idx3, loss_sum = _argmin_call(xf, frozen_codebook, W)
    idx_flat = idx3.reshape(M)
    quantized = _sc_gather(ic, idx_flat).reshape(B, N, DIM)
    indices = idx3.reshape(B, N)
    loss = loss_sum[0, 0] * LOSS_SCALE
    return quantized, indices, loss


# speedup vs baseline: 1.3871x; 1.3871x over previous
import functools

import jax
import jax.numpy as jnp
from jax import lax
from jax.experimental import pallas as pl
from jax.experimental.pallas import tpu as pltpu
from jax.experimental.pallas import tpu_sc as plsc

B, N, DIM = 8, 1024, 32
K = 8192
M = B * N
TN = 256
NUM_TILES = M // TN
LOSS_SCALE = 1.25 / (M * DIM)


def _ic_body(frozen_ref, w_ref, ic_ref):
    ic_ref[...] = jnp.dot(frozen_ref[...].astype(jnp.bfloat16),
                          w_ref[...].astype(jnp.bfloat16),
                          preferred_element_type=jnp.float32)


_ic_call = pl.pallas_call(
    _ic_body,
    out_shape=jax.ShapeDtypeStruct((K, DIM), jnp.float32),
)


def _argmin_body(xb_ref, a2_ref, b2_ref, icbt_ref, idx_ref, loss_ref):
    pi = pl.program_id(0)
    s = jnp.dot(xb_ref[...], icbt_ref[...],
                preferred_element_type=jnp.float32)
    d2 = (a2_ref[...] - 2.0 * s) + b2_ref[...]
    d2 = jnp.maximum(d2, 0.0)
    minv = jnp.min(d2, axis=1, keepdims=True)
    iota = lax.broadcasted_iota(jnp.int32, (TN, K), 1)
    idx = jnp.min(jnp.where(d2 == minv, iota, K), axis=1)
    idx_ref[0, 0, :] = idx
    part = jnp.sum(minv).reshape(1, 1)

    @pl.when(pi == 0)
    def _():
        loss_ref[...] = jnp.zeros((1, 1), jnp.float32)

    loss_ref[...] += part


_argmin_call = pl.pallas_call(
    _argmin_body,
    grid=(NUM_TILES,),
    in_specs=[
        pl.BlockSpec((TN, DIM), lambda i: (i, 0)),
        pl.BlockSpec((TN, 1), lambda i: (i, 0)),
        pl.BlockSpec((1, K), lambda i: (0, 0)),
        pl.BlockSpec((DIM, K), lambda i: (0, 0)),
    ],
    out_specs=[
        pl.BlockSpec((1, 1, TN), lambda i: (i, 0, 0)),
        pl.BlockSpec((1, 1), lambda i: (0, 0)),
    ],
    out_shape=[
        jax.ShapeDtypeStruct((NUM_TILES, 1, TN), jnp.int32),
        jax.ShapeDtypeStruct((1, 1), jnp.float32),
    ],
)


def _sc_gather(table, idx):
    info = plsc.get_sparse_core_info()
    nw = info.num_cores * info.num_subcores
    bpw = M // nw
    mesh = plsc.VectorSubcoreMesh(core_axis_name="c", subcore_axis_name="s")

    @functools.partial(
        pl.kernel, mesh=mesh,
        compiler_params=pltpu.CompilerParams(use_tc_tiling_on_sc=False),
        out_type=jax.ShapeDtypeStruct((M, DIM), jnp.float32),
        scratch_types=[
            pltpu.VMEM((bpw,), jnp.int32),
            pltpu.VMEM((bpw, DIM), jnp.float32),
            pltpu.SemaphoreType.DMA,
        ],
    )
    def k(table_hbm, idx_hbm, out_hbm, idx_v, rows_v, sem):
        wid = lax.axis_index("s") * info.num_cores + lax.axis_index("c")
        base = wid * bpw
        pltpu.sync_copy(idx_hbm.at[pl.ds(base, bpw)], idx_v)
        pltpu.async_copy(table_hbm.at[idx_v], rows_v, sem).wait()
        pltpu.sync_copy(rows_v, out_hbm.at[pl.ds(base, bpw)])

    return k(table, idx)


def kernel(x, frozen_codebook, W):
    xf = x.reshape(M, DIM)
    ic = _ic_call(frozen_codebook, W)
    a2 = jnp.sum(xf * xf, axis=-1, keepdims=True)
    b2 = jnp.sum(ic * ic, axis=-1)[None, :]
    xb = xf.astype(jnp.bfloat16)
    icbt = ic.astype(jnp.bfloat16).T
    idx3, loss_sum = _argmin_call(xb, a2, b2, icbt)
    idx_flat = idx3.reshape(M)
    quantized = _sc_gather(ic, idx_flat).reshape(B, N, DIM)
    indices = idx3.reshape(B, N)
    loss = loss_sum[0, 0] * LOSS_SCALE
    return quantized, indices, loss


# speedup vs baseline: 1.5168x; 1.0935x over previous
import functools

import jax
import jax.numpy as jnp
from jax import lax
from jax.experimental import pallas as pl
from jax.experimental.pallas import tpu as pltpu
from jax.experimental.pallas import tpu_sc as plsc

B, N, DIM = 8, 1024, 32
K = 8192
M = B * N
TN = 256
NUM_TILES = M // TN
LOSS_SCALE = 1.25 / (M * DIM)


def _ic_body(frozen_ref, w_ref, ic_ref):
    ic_ref[...] = jnp.dot(frozen_ref[...].astype(jnp.bfloat16),
                          w_ref[...].astype(jnp.bfloat16),
                          preferred_element_type=jnp.float32)


_ic_call = pl.pallas_call(
    _ic_body,
    out_shape=jax.ShapeDtypeStruct((K, DIM), jnp.float32),
)


CK = 128
NCK = K // CK


def _argmin_body(xb_ref, a2_ref, b2_ref, icbt_ref, idx_ref, loss_ref):
    pi = pl.program_id(0)
    s = jnp.dot(xb_ref[...], icbt_ref[...],
                preferred_element_type=jnp.float32)
    a2 = a2_ref[...]
    runmin = jnp.full((TN, CK), jnp.inf, jnp.float32)
    runidx = jnp.zeros((TN, CK), jnp.int32)
    for j in range(NCK):
        d2c = (a2 - 2.0 * s[:, j * CK:(j + 1) * CK]) + b2_ref[:, j * CK:(j + 1) * CK]
        d2c = jnp.maximum(d2c, 0.0)
        cmp = d2c < runmin
        runmin = jnp.where(cmp, d2c, runmin)
        runidx = jnp.where(cmp, j, runidx)
    minv = jnp.min(runmin, axis=1, keepdims=True)
    cand = runidx * CK + lax.broadcasted_iota(jnp.int32, (TN, CK), 1)
    idx = jnp.min(jnp.where(runmin == minv, cand, K), axis=1)
    idx_ref[0, 0, :] = idx
    part = jnp.sum(minv).reshape(1, 1)

    @pl.when(pi == 0)
    def _():
        loss_ref[...] = jnp.zeros((1, 1), jnp.float32)

    loss_ref[...] += part


_argmin_call = pl.pallas_call(
    _argmin_body,
    grid=(NUM_TILES,),
    in_specs=[
        pl.BlockSpec((TN, DIM), lambda i: (i, 0)),
        pl.BlockSpec((TN, 1), lambda i: (i, 0)),
        pl.BlockSpec((1, K), lambda i: (0, 0)),
        pl.BlockSpec((DIM, K), lambda i: (0, 0)),
    ],
    out_specs=[
        pl.BlockSpec((1, 1, TN), lambda i: (i, 0, 0)),
        pl.BlockSpec((1, 1), lambda i: (0, 0)),
    ],
    out_shape=[
        jax.ShapeDtypeStruct((NUM_TILES, 1, TN), jnp.int32),
        jax.ShapeDtypeStruct((1, 1), jnp.float32),
    ],
)


def _sc_gather(table, idx):
    info = plsc.get_sparse_core_info()
    nw = info.num_cores * info.num_subcores
    bpw = M // nw
    mesh = plsc.VectorSubcoreMesh(core_axis_name="c", subcore_axis_name="s")

    @functools.partial(
        pl.kernel, mesh=mesh,
        compiler_params=pltpu.CompilerParams(use_tc_tiling_on_sc=False),
        out_type=jax.ShapeDtypeStruct((M, DIM), jnp.float32),
        scratch_types=[
            pltpu.VMEM((bpw,), jnp.int32),
            pltpu.VMEM((bpw, DIM), jnp.float32),
            pltpu.SemaphoreType.DMA,
        ],
    )
    def k(table_hbm, idx_hbm, out_hbm, idx_v, rows_v, sem):
        wid = lax.axis_index("s") * info.num_cores + lax.axis_index("c")
        base = wid * bpw
        pltpu.sync_copy(idx_hbm.at[pl.ds(base, bpw)], idx_v)
        pltpu.async_copy(table_hbm.at[idx_v], rows_v, sem).wait()
        pltpu.sync_copy(rows_v, out_hbm.at[pl.ds(base, bpw)])

    return k(table, idx)


def kernel(x, frozen_codebook, W):
    xf = x.reshape(M, DIM)
    ic = _ic_call(frozen_codebook, W)
    a2 = jnp.sum(xf * xf, axis=-1, keepdims=True)
    b2 = jnp.sum(ic * ic, axis=-1)[None, :]
    xb = xf.astype(jnp.bfloat16)
    icbt = ic.astype(jnp.bfloat16).T
    idx3, loss_sum = _argmin_call(xb, a2, b2, icbt)
    idx_flat = idx3.reshape(M)
    quantized = _sc_gather(ic, idx_flat).reshape(B, N, DIM)
    indices = idx3.reshape(B, N)
    loss = loss_sum[0, 0] * LOSS_SCALE
    return quantized, indices, loss


# speedup vs baseline: 1.6101x; 1.0615x over previous
import functools

import jax
import jax.numpy as jnp
from jax import lax
from jax.experimental import pallas as pl
from jax.experimental.pallas import tpu as pltpu
from jax.experimental.pallas import tpu_sc as plsc

B, N, DIM = 8, 1024, 32
K = 8192
M = B * N
TN = 256
NUM_TILES = M // TN
LOSS_SCALE = 1.25 / (M * DIM)


def _ic_body(frozen_ref, w_ref, ic_ref):
    ic_ref[...] = jnp.dot(frozen_ref[...].astype(jnp.bfloat16),
                          w_ref[...].astype(jnp.bfloat16),
                          preferred_element_type=jnp.float32)


_ic_call = pl.pallas_call(
    _ic_body,
    out_shape=jax.ShapeDtypeStruct((K, DIM), jnp.float32),
)


CK = 128
NCK = K // CK


def _argmin_body(xb_ref, a2_ref, b2_ref, icbt_ref, idx_ref, loss_ref):
    pi = pl.program_id(0)
    s = jnp.dot(xb_ref[...], icbt_ref[...],
                preferred_element_type=jnp.float32)
    a2 = a2_ref[...]
    runmin = jnp.full((TN, CK), jnp.inf, jnp.float32)
    runidx = jnp.zeros((TN, CK), jnp.int32)
    for j in range(NCK):
        d2c = (a2 + s[:, j * CK:(j + 1) * CK]) + b2_ref[:, j * CK:(j + 1) * CK]
        d2c = jnp.maximum(d2c, 0.0)
        cmp = d2c < runmin
        runmin = jnp.where(cmp, d2c, runmin)
        runidx = jnp.where(cmp, j, runidx)
    minv = jnp.min(runmin, axis=1, keepdims=True)
    cand = runidx * CK + lax.broadcasted_iota(jnp.int32, (TN, CK), 1)
    idx = jnp.min(jnp.where(runmin == minv, cand, K), axis=1)
    idx_ref[0, 0, :] = idx
    loss_ref[...] = jnp.sum(minv).reshape(1, 1, 1)


_argmin_call = pl.pallas_call(
    _argmin_body,
    grid=(NUM_TILES,),
    in_specs=[
        pl.BlockSpec((TN, DIM), lambda i: (i, 0)),
        pl.BlockSpec((TN, 1), lambda i: (i, 0)),
        pl.BlockSpec((1, K), lambda i: (0, 0)),
        pl.BlockSpec((DIM, K), lambda i: (0, 0)),
    ],
    out_specs=[
        pl.BlockSpec((1, 1, TN), lambda i: (i, 0, 0)),
        pl.BlockSpec((1, 1, 1), lambda i: (i, 0, 0)),
    ],
    out_shape=[
        jax.ShapeDtypeStruct((NUM_TILES, 1, TN), jnp.int32),
        jax.ShapeDtypeStruct((NUM_TILES, 1, 1), jnp.float32),
    ],
    compiler_params=pltpu.CompilerParams(
        dimension_semantics=("parallel",)),
)


def _sc_gather(table, idx):
    info = plsc.get_sparse_core_info()
    nw = info.num_cores * info.num_subcores
    bpw = M // nw
    mesh = plsc.VectorSubcoreMesh(core_axis_name="c", subcore_axis_name="s")

    @functools.partial(
        pl.kernel, mesh=mesh,
        compiler_params=pltpu.CompilerParams(use_tc_tiling_on_sc=False),
        out_type=jax.ShapeDtypeStruct((M, DIM), jnp.float32),
        scratch_types=[
            pltpu.VMEM((bpw,), jnp.int32),
            pltpu.VMEM((bpw, DIM), jnp.float32),
            pltpu.SemaphoreType.DMA,
        ],
    )
    def k(table_hbm, idx_hbm, out_hbm, idx_v, rows_v, sem):
        wid = lax.axis_index("s") * info.num_cores + lax.axis_index("c")
        base = wid * bpw
        pltpu.sync_copy(idx_hbm.at[pl.ds(base, bpw)], idx_v)
        pltpu.async_copy(table_hbm.at[idx_v], rows_v, sem).wait()
        pltpu.sync_copy(rows_v, out_hbm.at[pl.ds(base, bpw)])

    return k(table, idx)


def kernel(x, frozen_codebook, W):
    xf = x.reshape(M, DIM)
    ic = _ic_call(frozen_codebook, W)
    a2 = jnp.sum(xf * xf, axis=-1, keepdims=True)
    b2 = jnp.sum(ic * ic, axis=-1)[None, :]
    xb = xf.astype(jnp.bfloat16)
    icbt = (ic * -2.0).astype(jnp.bfloat16).T
    idx3, loss_parts = _argmin_call(xb, a2, b2, icbt)
    idx_flat = idx3.reshape(M)
    quantized = _sc_gather(ic, idx_flat).reshape(B, N, DIM)
    indices = idx3.reshape(B, N)
    loss = jnp.sum(loss_parts) * LOSS_SCALE
    return quantized, indices, loss
